# baseline (speedup 1.0000x reference)
import functools

import numpy as np
import jax
import jax.numpy as jnp
from jax import lax
from jax.experimental import pallas as pl
from jax.experimental.pallas import tpu as pltpu
from jax.experimental.pallas import tpu_sc as plsc

N_NODES = 10000
N_EDGES = 320000
D_IN = 128
PHM_DIM = 4
D_OUT = 128

NC = 2
NS = 16
EH = N_EDGES // NC
NPC = 80
NCHUNKS = N_NODES // NPC
NPASS = (NCHUNKS + NS - 1) // NS
NJUNK = NPC
CH = 2000
NECH = EH // CH
NV = CH // 16
GB = 256
NQ = D_IN // 16

_DEG = np.array([16, 24, 32, 32, 32, 32, 40, 48], dtype=np.float32)
AVG_LOG = float(np.log(_DEG + 1.0).mean())
FBIG = float(3.0e38)


def _sc_aggregate(x, idx, eids, bounds):
    mesh = plsc.VectorSubcoreMesh(core_axis_name="c", subcore_axis_name="s")

    @functools.partial(
        pl.kernel,
        mesh=mesh,
        compiler_params=pltpu.CompilerParams(needs_layout_passes=False),
        out_type=[
            jax.ShapeDtypeStruct((NC, N_NODES, D_IN), jnp.float32),
            jax.ShapeDtypeStruct((NC, N_NODES, D_IN), jnp.float32),
            jax.ShapeDtypeStruct((NC, N_NODES, 16), jnp.float32),
            jax.ShapeDtypeStruct((NC, N_NODES, D_IN), jnp.float32),
            jax.ShapeDtypeStruct((NC, N_NODES, D_IN), jnp.float32),
        ],
        scratch_types=[
            pltpu.VMEM((CH,), jnp.int32),
            pltpu.VMEM((CH,), jnp.int32),
            pltpu.VMEM((16,), jnp.int32),
            pltpu.VMEM((16,), jnp.int32),
            pltpu.VMEM((CH,), jnp.int32),
            pltpu.VMEM((CH,), jnp.int32),
            pltpu.VMEM((GB, D_IN), jnp.float32),
            pltpu.VMEM((NPC + 1, D_IN), jnp.float32),
            pltpu.VMEM((NPC + 1, D_IN), jnp.float32),
            pltpu.VMEM((NPC + 1, D_IN), jnp.float32),
            pltpu.VMEM((NPC + 1, D_IN), jnp.float32),
            pltpu.VMEM((NPC + 1, 16), jnp.float32),
            pltpu.SemaphoreType.DMA,
        ],
    )
    def k(x_hbm, idx_hbm, eids_hbm, bounds_hbm, osum, osq, ocnt, omin, omax,
          ibuf, ebuf, bbuf, lbuf, listg, listn, xg, accsum, accsq, accmin,
          accmax, acccnt, sem):
        cid = lax.axis_index("c")
        sid = lax.axis_index("s")
        ebase = cid * EH

        zeros16 = jnp.zeros((16,), jnp.float32)
        big16 = jnp.full((16,), FBIG, jnp.float32)
        negbig16 = jnp.full((16,), -FBIG, jnp.float32)
        onef16 = jnp.full((16,), 1.0, jnp.float32)
        one16 = jnp.full((16,), 1, jnp.int32)
        zero16i = jnp.zeros((16,), jnp.int32)
        npc16 = jnp.full((16,), NPC, jnp.int32)
        junk16 = jnp.full((16,), NJUNK, jnp.int32)

        def pass_body(r, _):
            kk = r * NS + sid
            nbase = kk * NPC
            pltpu.sync_copy(bounds_hbm.at[kk], bbuf)
            nb16 = bbuf[pl.ds(0, 16)]
            lim16 = nb16 + npc16

            def arow(q, _):
                for j in range(NQ):
                    sl = pl.ds(j * 16, 16)
                    accsum[q, sl] = zeros16
                    accsq[q, sl] = zeros16
                    accmin[q, sl] = big16
                    accmax[q, sl] = negbig16
                acccnt[q, pl.ds(0, 16)] = zeros16
                return 0
            lax.fori_loop(0, NPC + 1, arow, 0)

            def chunk_body(cc, _):
                e0 = ebase + cc * CH
                pltpu.sync_copy(idx_hbm.at[pl.ds(e0, CH)], ibuf)
                pltpu.sync_copy(eids_hbm.at[pl.ds(e0, CH)], ebuf)

                def prefill(v, _):
                    listg[pl.ds(v * 16, 16)] = zero16i
                    listn[pl.ds(v * 16, 16)] = junk16
                    return 0
                lax.fori_loop(0, NV, prefill, 0)

                lbuf[pl.ds(0, 16)] = zero16i

                def scan_body(v, _):
                    loff16 = lbuf[pl.ds(0, 16)]
                    iv = ibuf[pl.ds(v * 16, 16)]
                    m = (iv >= nb16) & (iv < lim16)
                    mi = jnp.where(m, one16, zero16i)
                    cs = jnp.cumsum(mi)
                    pos = (loff16 + cs) - one16
                    eg = ebuf[pl.ds(v * 16, 16)]
                    plsc.store_scatter(listg, [pos], eg, mask=m)
                    plsc.store_scatter(listn, [pos], iv - nb16, mask=m)
                    lbuf[pl.ds(0, 16)] = (
                        loff16 + plsc.all_reduce_population_count(m))
                    return 0
                lax.fori_loop(0, NV, scan_body, 0)
                nmatch = lbuf[pl.ds(0, 16)][0]

                def batch_body(b, _):
                    pltpu.async_copy(
                        x_hbm.at[listg.at[pl.ds(b * GB, GB)]], xg, sem,
                    ).wait()

                    def group_body(g, _):
                        nvec = listn[pl.ds(b * GB + g * 16, 16)]
                        for l in range(16):
                            mrow = g * 16 + l
                            n = nvec[l]
                            cv = acccnt[n, pl.ds(0, 16)]
                            acccnt[n, pl.ds(0, 16)] = cv + onef16
                            for j in range(NQ):
                                sl = pl.ds(j * 16, 16)
                                xv = xg[mrow, sl]
                                accsum[n, sl] = accsum[n, sl] + xv
                                accsq[n, sl] = accsq[n, sl] + xv * xv
                                accmin[n, sl] = jnp.minimum(accmin[n, sl], xv)
                                accmax[n, sl] = jnp.maximum(accmax[n, sl], xv)
                        return 0
                    lax.fori_loop(0, GB // 16, group_body, 0)
                    return 0
                lax.fori_loop(0, (nmatch + GB - 1) // GB, batch_body, 0)
                return 0
            lax.fori_loop(0, NECH, chunk_body, 0)

            @pl.when(kk < NCHUNKS)
            def _():
                rows = pl.ds(nbase, NPC)
                src = pl.ds(0, NPC)
                pltpu.sync_copy(accsum.at[src], osum.at[cid, rows])
                pltpu.sync_copy(accsq.at[src], osq.at[cid, rows])
                pltpu.sync_copy(acccnt.at[src], ocnt.at[cid, rows])
                pltpu.sync_copy(accmin.at[src], omin.at[cid, rows])
                pltpu.sync_copy(accmax.at[src], omax.at[cid, rows])
            return 0
        lax.fori_loop(0, NPASS, pass_body, 0)

    return k(x, idx, eids, bounds)


TN = 400


def _epilogue_body(s0_ref, s1_ref, q0_ref, q1_ref, mn0_ref, mn1_ref,
                   mx0_ref, mx1_ref, c0_ref, c1_ref, pr_ref, w_ref,
                   b_ref, o_ref):
    s = s0_ref[...] + s1_ref[...]
    sq = q0_ref[...] + q1_ref[...]
    cnt = (c0_ref[...] + c1_ref[...])[:, 0:1]
    safe = jnp.maximum(cnt, 1.0)
    mean = s / safe
    var = sq / safe - mean * mean
    std = jnp.sqrt(jax.nn.relu(var) + 1e-05)
    has = cnt > 0.0
    mn = jnp.where(has, jnp.minimum(mn0_ref[...], mn1_ref[...]), 0.0)
    mx = jnp.where(has, jnp.maximum(mx0_ref[...], mx1_ref[...]), 0.0)

    parts1 = []
    for p in range(PHM_DIM):
        sl = slice(32 * p, 32 * p + 32)
        parts1 += [mean[:, sl], mn[:, sl], mx[:, sl], std[:, sl]]
    out = jnp.concatenate(parts1, axis=1)

    logdeg = jnp.log(cnt + 1.0)
    amp_s = logdeg / AVG_LOG
    att_s = jnp.where(cnt == 0.0, 1.0,
                      AVG_LOG / jnp.where(cnt == 0.0, 1.0, logdeg))

    parts2 = []
    for p in range(PHM_DIM):
        sl = slice(128 * p, 128 * p + 128)
        op = out[:, sl]
        parts2 += [op, op * amp_s, op * att_s]
    big = jnp.concatenate(parts2, axis=1)

    pr = pr_ref[...]
    w = w_ref[...]
    row_blocks = []
    for p in range(PHM_DIM):
        col_blocks = []
        for j in range(PHM_DIM):
            blk = pr[0, p, j] * w[0]
            for i in range(1, PHM_DIM):
                blk = blk + pr[i, p, j] * w[i]
            col_blocks.append(blk)
        row_blocks.append(jnp.concatenate(col_blocks, axis=1))
    H = jnp.concatenate(row_blocks, axis=0)

    o_ref[...] = (jnp.dot(big, H, preferred_element_type=jnp.float32)
                  + b_ref[...])


def _epilogue(osum, osq, ocnt, omin, omax, phm_rule, W, bias):
    grid = (N_NODES // TN,)
    node_spec = pl.BlockSpec((TN, D_IN), lambda i: (i, 0))
    cnt_spec = pl.BlockSpec((TN, 16), lambda i: (i, 0))
    return pl.pallas_call(
        _epilogue_body,
        grid=grid,
        in_specs=[
            node_spec, node_spec, node_spec, node_spec,
            node_spec, node_spec, node_spec, node_spec,
            cnt_spec, cnt_spec,
            pl.BlockSpec((PHM_DIM, PHM_DIM, PHM_DIM), lambda i: (0, 0, 0)),
            pl.BlockSpec((PHM_DIM, 384, 32), lambda i: (0, 0, 0)),
            pl.BlockSpec((1, D_OUT), lambda i: (0, 0)),
        ],
        out_specs=pl.BlockSpec((TN, D_OUT), lambda i: (i, 0)),
        out_shape=jax.ShapeDtypeStruct((N_NODES, D_OUT), jnp.float32),
    )(osum[0], osum[1], osq[0], osq[1], omin[0], omin[1], omax[0], omax[1],
      ocnt[0], ocnt[1], phm_rule, W, bias)


def kernel(x, idx, phm_rule, W, bias, dim_size):
    eids = jnp.arange(N_EDGES, dtype=jnp.int32)
    bounds = jnp.broadcast_to(
        (jnp.arange(NS * NPASS, dtype=jnp.int32) * NPC)[:, None],
        (NS * NPASS, 16)).astype(jnp.int32)
    osum, osq, ocnt, omin, omax = _sc_aggregate(x, idx, eids, bounds)
    return _epilogue(osum, osq, ocnt, omin, omax, phm_rule, W,
                     bias.reshape(1, D_OUT))


# speedup vs baseline: 14.4512x; 14.4512x over previous
import functools

import numpy as np
import jax
import jax.numpy as jnp
from jax import lax
from jax.experimental import pallas as pl
from jax.experimental.pallas import tpu as pltpu
from jax.experimental.pallas import tpu_sc as plsc

N_NODES = 10000
N_EDGES = 320000
D_IN = 128
PHM_DIM = 4
D_OUT = 128

NC = 2
NS = 16
EH = N_EDGES // NC
NPC = 80
NCHUNKS = N_NODES // NPC
NPASS = (NCHUNKS + NS - 1) // NS
NJUNK = NPC
CH = 2000
NECH = EH // CH
NV = CH // 16
GB = 32
NQ = D_IN // 16

_DEG = np.array([16, 24, 32, 32, 32, 32, 40, 48], dtype=np.float32)
AVG_LOG = float(np.log(_DEG + 1.0).mean())
FBIG = float(3.0e38)


def _sc_aggregate(x, idx, eids, bounds):
    mesh = plsc.VectorSubcoreMesh(core_axis_name="c", subcore_axis_name="s")

    @functools.partial(
        pl.kernel,
        mesh=mesh,
        compiler_params=pltpu.CompilerParams(needs_layout_passes=False),
        out_type=[
            jax.ShapeDtypeStruct((NC, N_NODES, D_IN), jnp.float32),
            jax.ShapeDtypeStruct((NC, N_NODES, D_IN), jnp.float32),
            jax.ShapeDtypeStruct((NC, N_NODES, 16), jnp.float32),
            jax.ShapeDtypeStruct((NC, N_NODES, D_IN), jnp.float32),
            jax.ShapeDtypeStruct((NC, N_NODES, D_IN), jnp.float32),
        ],
        scratch_types=[
            pltpu.VMEM((CH,), jnp.int32),
            pltpu.VMEM((CH,), jnp.int32),
            pltpu.VMEM((16,), jnp.int32),
            pltpu.VMEM((16,), jnp.int32),
            pltpu.VMEM((CH,), jnp.int32),
            pltpu.VMEM((CH,), jnp.int32),
            pltpu.VMEM((GB, D_IN), jnp.float32),
            pltpu.VMEM((NPC + 1, D_IN), jnp.float32),
            pltpu.VMEM((NPC + 1, D_IN), jnp.float32),
            pltpu.VMEM((NPC + 1, D_IN), jnp.float32),
            pltpu.VMEM((NPC + 1, D_IN), jnp.float32),
            pltpu.VMEM((NPC + 1, 16), jnp.float32),
            pltpu.SemaphoreType.DMA,
        ],
    )
    def k(x_hbm, idx_hbm, eids_hbm, bounds_hbm, osum, osq, ocnt, omin, omax,
          ibuf, ebuf, bbuf, lbuf, listg, listn, xg, accsum, accsq, accmin,
          accmax, acccnt, sem):
        cid = lax.axis_index("c")
        sid = lax.axis_index("s")
        ebase = cid * EH

        zeros16 = jnp.zeros((16,), jnp.float32)
        big16 = jnp.full((16,), FBIG, jnp.float32)
        negbig16 = jnp.full((16,), -FBIG, jnp.float32)
        onef16 = jnp.full((16,), 1.0, jnp.float32)
        one16 = jnp.full((16,), 1, jnp.int32)
        zero16i = jnp.zeros((16,), jnp.int32)
        npc16 = jnp.full((16,), NPC, jnp.int32)
        junk16 = jnp.full((16,), NJUNK, jnp.int32)

        def pass_body(r, _):
            kk = r * NS + sid
            nbase = kk * NPC
            pltpu.sync_copy(bounds_hbm.at[kk], bbuf)
            nb16 = bbuf[pl.ds(0, 16)]
            lim16 = nb16 + npc16

            def arow(q, _):
                for j in range(NQ):
                    sl = pl.ds(j * 16, 16)
                    accsum[q, sl] = zeros16
                    accsq[q, sl] = zeros16
                    accmin[q, sl] = big16
                    accmax[q, sl] = negbig16
                acccnt[q, pl.ds(0, 16)] = zeros16
                return 0
            lax.fori_loop(0, NPC + 1, arow, 0)

            def chunk_body(cc, _):
                e0 = ebase + cc * CH
                pltpu.sync_copy(idx_hbm.at[pl.ds(e0, CH)], ibuf)
                pltpu.sync_copy(eids_hbm.at[pl.ds(e0, CH)], ebuf)

                def prefill(v, _):
                    listg[pl.ds(v * 16, 16)] = zero16i
                    listn[pl.ds(v * 16, 16)] = junk16
                    return 0
                lax.fori_loop(0, NV, prefill, 0)

                lbuf[pl.ds(0, 16)] = zero16i

                def scan_body(v, _):
                    loff16 = lbuf[pl.ds(0, 16)]
                    iv = ibuf[pl.ds(v * 16, 16)]
                    m = (iv >= nb16) & (iv < lim16)
                    mi = jnp.where(m, one16, zero16i)
                    cs = jnp.cumsum(mi)
                    pos = (loff16 + cs) - one16
                    eg = ebuf[pl.ds(v * 16, 16)]
                    plsc.store_scatter(listg, [pos], eg, mask=m)
                    plsc.store_scatter(listn, [pos], iv - nb16, mask=m)
                    lbuf[pl.ds(0, 16)] = (
                        loff16 + plsc.all_reduce_population_count(m))
                    return 0
                lax.fori_loop(0, NV, scan_body, 0)
                nmatch = lbuf[pl.ds(0, 16)][0]

                def batch_body(b, _):
                    pltpu.async_copy(
                        x_hbm.at[listg.at[pl.ds(b * GB, GB)]], xg, sem,
                    ).wait()

                    def group_body(g, _):
                        nvec = listn[pl.ds(b * GB + g * 16, 16)]
                        for l in range(16):
                            mrow = g * 16 + l
                            n = nvec[l]
                            cv = acccnt[n, pl.ds(0, 16)]
                            acccnt[n, pl.ds(0, 16)] = cv + onef16
                            for j in range(NQ):
                                sl = pl.ds(j * 16, 16)
                                xv = xg[mrow, sl]
                                accsum[n, sl] = accsum[n, sl] + xv
                                accsq[n, sl] = accsq[n, sl] + xv * xv
                                accmin[n, sl] = jnp.minimum(accmin[n, sl], xv)
                                accmax[n, sl] = jnp.maximum(accmax[n, sl], xv)
                        return 0
                    lax.fori_loop(0, GB // 16, group_body, 0)
                    return 0
                lax.fori_loop(0, (nmatch + GB - 1) // GB, batch_body, 0)
                return 0
            lax.fori_loop(0, NECH, chunk_body, 0)

            @pl.when(kk < NCHUNKS)
            def _():
                rows = pl.ds(nbase, NPC)
                src = pl.ds(0, NPC)
                pltpu.sync_copy(accsum.at[src], osum.at[cid, rows])
                pltpu.sync_copy(accsq.at[src], osq.at[cid, rows])
                pltpu.sync_copy(acccnt.at[src], ocnt.at[cid, rows])
                pltpu.sync_copy(accmin.at[src], omin.at[cid, rows])
                pltpu.sync_copy(accmax.at[src], omax.at[cid, rows])
            return 0
        lax.fori_loop(0, NPASS, pass_body, 0)

    return k(x, idx, eids, bounds)


TN = 400


def _epilogue_body(s0_ref, s1_ref, q0_ref, q1_ref, mn0_ref, mn1_ref,
                   mx0_ref, mx1_ref, c0_ref, c1_ref, pr_ref, w_ref,
                   b_ref, o_ref):
    s = s0_ref[...] + s1_ref[...]
    sq = q0_ref[...] + q1_ref[...]
    cnt = (c0_ref[...] + c1_ref[...])[:, 0:1]
    safe = jnp.maximum(cnt, 1.0)
    mean = s / safe
    var = sq / safe - mean * mean
    std = jnp.sqrt(jax.nn.relu(var) + 1e-05)
    has = cnt > 0.0
    mn = jnp.where(has, jnp.minimum(mn0_ref[...], mn1_ref[...]), 0.0)
    mx = jnp.where(has, jnp.maximum(mx0_ref[...], mx1_ref[...]), 0.0)

    parts1 = []
    for p in range(PHM_DIM):
        sl = slice(32 * p, 32 * p + 32)
        parts1 += [mean[:, sl], mn[:, sl], mx[:, sl], std[:, sl]]
    out = jnp.concatenate(parts1, axis=1)

    logdeg = jnp.log(cnt + 1.0)
    amp_s = logdeg / AVG_LOG
    att_s = jnp.where(cnt == 0.0, 1.0,
                      AVG_LOG / jnp.where(cnt == 0.0, 1.0, logdeg))

    parts2 = []
    for p in range(PHM_DIM):
        sl = slice(128 * p, 128 * p + 128)
        op = out[:, sl]
        parts2 += [op, op * amp_s, op * att_s]
    big = jnp.concatenate(parts2, axis=1)

    pr = pr_ref[...]
    w = w_ref[...]
    row_blocks = []
    for p in range(PHM_DIM):
        col_blocks = []
        for j in range(PHM_DIM):
            blk = pr[0, p, j] * w[0]
            for i in range(1, PHM_DIM):
                blk = blk + pr[i, p, j] * w[i]
            col_blocks.append(blk)
        row_blocks.append(jnp.concatenate(col_blocks, axis=1))
    H = jnp.concatenate(row_blocks, axis=0)

    o_ref[...] = (jnp.dot(big, H, preferred_element_type=jnp.float32)
                  + b_ref[...])


def _epilogue(osum, osq, ocnt, omin, omax, phm_rule, W, bias):
    grid = (N_NODES // TN,)
    node_spec = pl.BlockSpec((TN, D_IN), lambda i: (i, 0))
    cnt_spec = pl.BlockSpec((TN, 16), lambda i: (i, 0))
    return pl.pallas_call(
        _epilogue_body,
        grid=grid,
        in_specs=[
            node_spec, node_spec, node_spec, node_spec,
            node_spec, node_spec, node_spec, node_spec,
            cnt_spec, cnt_spec,
            pl.BlockSpec((PHM_DIM, PHM_DIM, PHM_DIM), lambda i: (0, 0, 0)),
            pl.BlockSpec((PHM_DIM, 384, 32), lambda i: (0, 0, 0)),
            pl.BlockSpec((1, D_OUT), lambda i: (0, 0)),
        ],
        out_specs=pl.BlockSpec((TN, D_OUT), lambda i: (i, 0)),
        out_shape=jax.ShapeDtypeStruct((N_NODES, D_OUT), jnp.float32),
    )(osum[0], osum[1], osq[0], osq[1], omin[0], omin[1], omax[0], omax[1],
      ocnt[0], ocnt[1], phm_rule, W, bias)


def kernel(x, idx, phm_rule, W, bias, dim_size):
    eids = jnp.arange(N_EDGES, dtype=jnp.int32)
    bounds = jnp.broadcast_to(
        (jnp.arange(NS * NPASS, dtype=jnp.int32) * NPC)[:, None],
        (NS * NPASS, 16)).astype(jnp.int32)
    osum, osq, ocnt, omin, omax = _sc_aggregate(x, idx, eids, bounds)
    return _epilogue(osum, osq, ocnt, omin, omax, phm_rule, W,
                     bias.reshape(1, D_OUT))


# speedup vs baseline: 54.1912x; 3.7499x over previous
import functools

import numpy as np
import jax
import jax.numpy as jnp
from jax import lax
from jax.experimental import pallas as pl
from jax.experimental.pallas import tpu as pltpu
from jax.experimental.pallas import tpu_sc as plsc

N_NODES = 10000
N_EDGES = 320000
D_IN = 128
PHM_DIM = 4
D_OUT = 128

NC = 2
NS = 16
EH = N_EDGES // NC
NPC = 80
NCHUNKS = N_NODES // NPC
NPASS = (NCHUNKS + NS - 1) // NS
NJUNK = NPC
CH = 8000
NECH = EH // CH
NV = CH // 16
GB = 32
NQ = D_IN // 16

_DEG = np.array([16, 24, 32, 32, 32, 32, 40, 48], dtype=np.float32)
AVG_LOG = float(np.log(_DEG + 1.0).mean())
FBIG = float(3.0e38)


def _sc_aggregate(x, idx, eids, bounds):
    mesh = plsc.VectorSubcoreMesh(core_axis_name="c", subcore_axis_name="s")

    @functools.partial(
        pl.kernel,
        mesh=mesh,
        compiler_params=pltpu.CompilerParams(needs_layout_passes=False),
        out_type=[
            jax.ShapeDtypeStruct((NC, N_NODES, D_IN), jnp.float32),
            jax.ShapeDtypeStruct((NC, N_NODES, D_IN), jnp.float32),
            jax.ShapeDtypeStruct((NC, N_NODES, 16), jnp.float32),
            jax.ShapeDtypeStruct((NC, N_NODES, D_IN), jnp.float32),
            jax.ShapeDtypeStruct((NC, N_NODES, D_IN), jnp.float32),
        ],
        scratch_types=[
            pltpu.VMEM((CH,), jnp.int32),
            pltpu.VMEM((16,), jnp.int32),
            pltpu.VMEM((16,), jnp.int32),
            pltpu.VMEM((CH + GB,), jnp.int32),
            pltpu.VMEM((CH + GB,), jnp.int32),
            pltpu.VMEM((GB, D_IN), jnp.float32),
            pltpu.VMEM((NPC + 1, D_IN), jnp.float32),
            pltpu.VMEM((NPC + 1, D_IN), jnp.float32),
            pltpu.VMEM((NPC + 1, D_IN), jnp.float32),
            pltpu.VMEM((NPC + 1, D_IN), jnp.float32),
            pltpu.VMEM((NPC + 1, 16), jnp.float32),
            pltpu.SemaphoreType.DMA,
        ],
    )
    def k(x_hbm, idx_hbm, eids_hbm, bounds_hbm, osum, osq, ocnt, omin, omax,
          ibuf, bbuf, lbuf, listg, listn, xg, accsum, accsq, accmin,
          accmax, acccnt, sem):
        cid = lax.axis_index("c")
        sid = lax.axis_index("s")
        ebase = cid * EH

        zeros16 = jnp.zeros((16,), jnp.float32)
        big16 = jnp.full((16,), FBIG, jnp.float32)
        negbig16 = jnp.full((16,), -FBIG, jnp.float32)
        onef16 = jnp.full((16,), 1.0, jnp.float32)
        one16 = jnp.full((16,), 1, jnp.int32)
        lane = lax.iota(jnp.int32, 16)
        zero16i = jnp.zeros((16,), jnp.int32)
        npc16 = jnp.full((16,), NPC, jnp.int32)
        junk16 = jnp.full((16,), NJUNK, jnp.int32)

        def pass_body(r, _):
            kk = r * NS + sid
            nbase = kk * NPC
            pltpu.sync_copy(bounds_hbm.at[kk], bbuf)
            nb16 = bbuf[pl.ds(0, 16)]
            lim16 = nb16 + npc16

            def arow(q, _):
                for j in range(NQ):
                    sl = pl.ds(j * 16, 16)
                    accsum[q, sl] = zeros16
                    accsq[q, sl] = zeros16
                    accmin[q, sl] = big16
                    accmax[q, sl] = negbig16
                acccnt[q, pl.ds(0, 16)] = zeros16
                return 0
            lax.fori_loop(0, NPC + 1, arow, 0)

            def chunk_body(cc, _):
                e0 = ebase + cc * CH
                pltpu.sync_copy(idx_hbm.at[pl.ds(e0, CH)], ibuf)

                lbuf[pl.ds(0, 16)] = zero16i

                def scan_body(v, _):
                    loff16 = lbuf[pl.ds(0, 16)]
                    iv = ibuf[pl.ds(v * 16, 16)]
                    m = (iv >= nb16) & (iv < lim16)
                    mi = jnp.where(m, one16, zero16i)
                    cs = jnp.cumsum(mi)
                    pos = (loff16 + cs) - one16
                    eg = jnp.full((16,), e0 + v * 16, jnp.int32) + lane
                    plsc.store_scatter(listg, [pos], eg, mask=m)
                    plsc.store_scatter(listn, [pos], iv - nb16, mask=m)
                    lbuf[pl.ds(0, 16)] = (
                        loff16 + plsc.all_reduce_population_count(m))
                    return 0
                lax.fori_loop(0, NV, scan_body, 0)
                nmatch = lbuf[pl.ds(0, 16)][0]

                for t in range(GB // 16):
                    padpos = jnp.full((16,), nmatch + t * 16, jnp.int32) + lane
                    plsc.store_scatter(listg, [padpos], zero16i)
                    plsc.store_scatter(listn, [padpos], junk16)

                def batch_body(b, _):
                    pltpu.async_copy(
                        x_hbm.at[listg.at[pl.ds(b * GB, GB)]], xg, sem,
                    ).wait()

                    def group_body(g, _):
                        nvec = listn[pl.ds(b * GB + g * 16, 16)]
                        for l in range(16):
                            mrow = g * 16 + l
                            n = nvec[l]
                            cv = acccnt[n, pl.ds(0, 16)]
                            acccnt[n, pl.ds(0, 16)] = cv + onef16
                            for j in range(NQ):
                                sl = pl.ds(j * 16, 16)
                                xv = xg[mrow, sl]
                                accsum[n, sl] = accsum[n, sl] + xv
                                accsq[n, sl] = accsq[n, sl] + xv * xv
                                accmin[n, sl] = jnp.minimum(accmin[n, sl], xv)
                                accmax[n, sl] = jnp.maximum(accmax[n, sl], xv)
                        return 0
                    lax.fori_loop(0, GB // 16, group_body, 0)
                    return 0
                lax.fori_loop(0, (nmatch + GB - 1) // GB, batch_body, 0)
                return 0
            lax.fori_loop(0, NECH, chunk_body, 0)

            @pl.when(kk < NCHUNKS)
            def _():
                rows = pl.ds(nbase, NPC)
                src = pl.ds(0, NPC)
                pltpu.sync_copy(accsum.at[src], osum.at[cid, rows])
                pltpu.sync_copy(accsq.at[src], osq.at[cid, rows])
                pltpu.sync_copy(acccnt.at[src], ocnt.at[cid, rows])
                pltpu.sync_copy(accmin.at[src], omin.at[cid, rows])
                pltpu.sync_copy(accmax.at[src], omax.at[cid, rows])
            return 0
        lax.fori_loop(0, NPASS, pass_body, 0)

    return k(x, idx, eids, bounds)


TN = 400


def _epilogue_body(s0_ref, s1_ref, q0_ref, q1_ref, mn0_ref, mn1_ref,
                   mx0_ref, mx1_ref, c0_ref, c1_ref, pr_ref, w_ref,
                   b_ref, o_ref):
    s = s0_ref[...] + s1_ref[...]
    sq = q0_ref[...] + q1_ref[...]
    cnt = (c0_ref[...] + c1_ref[...])[:, 0:1]
    safe = jnp.maximum(cnt, 1.0)
    mean = s / safe
    var = sq / safe - mean * mean
    std = jnp.sqrt(jax.nn.relu(var) + 1e-05)
    has = cnt > 0.0
    mn = jnp.where(has, jnp.minimum(mn0_ref[...], mn1_ref[...]), 0.0)
    mx = jnp.where(has, jnp.maximum(mx0_ref[...], mx1_ref[...]), 0.0)

    parts1 = []
    for p in range(PHM_DIM):
        sl = slice(32 * p, 32 * p + 32)
        parts1 += [mean[:, sl], mn[:, sl], mx[:, sl], std[:, sl]]
    out = jnp.concatenate(parts1, axis=1)

    logdeg = jnp.log(cnt + 1.0)
    amp_s = logdeg / AVG_LOG
    att_s = jnp.where(cnt == 0.0, 1.0,
                      AVG_LOG / jnp.where(cnt == 0.0, 1.0, logdeg))

    parts2 = []
    for p in range(PHM_DIM):
        sl = slice(128 * p, 128 * p + 128)
        op = out[:, sl]
        parts2 += [op, op * amp_s, op * att_s]
    big = jnp.concatenate(parts2, axis=1)

    pr = pr_ref[...]
    w = w_ref[...]
    row_blocks = []
    for p in range(PHM_DIM):
        col_blocks = []
        for j in range(PHM_DIM):
            blk = pr[0, p, j] * w[0]
            for i in range(1, PHM_DIM):
                blk = blk + pr[i, p, j] * w[i]
            col_blocks.append(blk)
        row_blocks.append(jnp.concatenate(col_blocks, axis=1))
    H = jnp.concatenate(row_blocks, axis=0)

    o_ref[...] = (jnp.dot(big, H, preferred_element_type=jnp.float32)
                  + b_ref[...])


def _epilogue(osum, osq, ocnt, omin, omax, phm_rule, W, bias):
    grid = (N_NODES // TN,)
    node_spec = pl.BlockSpec((TN, D_IN), lambda i: (i, 0))
    cnt_spec = pl.BlockSpec((TN, 16), lambda i: (i, 0))
    return pl.pallas_call(
        _epilogue_body,
        grid=grid,
        in_specs=[
            node_spec, node_spec, node_spec, node_spec,
            node_spec, node_spec, node_spec, node_spec,
            cnt_spec, cnt_spec,
            pl.BlockSpec((PHM_DIM, PHM_DIM, PHM_DIM), lambda i: (0, 0, 0)),
            pl.BlockSpec((PHM_DIM, 384, 32), lambda i: (0, 0, 0)),
            pl.BlockSpec((1, D_OUT), lambda i: (0, 0)),
        ],
        out_specs=pl.BlockSpec((TN, D_OUT), lambda i: (i, 0)),
        out_shape=jax.ShapeDtypeStruct((N_NODES, D_OUT), jnp.float32),
    )(osum[0], osum[1], osq[0], osq[1], omin[0], omin[1], omax[0], omax[1],
      ocnt[0], ocnt[1], phm_rule, W, bias)


def kernel(x, idx, phm_rule, W, bias, dim_size):
    eids = jnp.arange(N_EDGES, dtype=jnp.int32)
    bounds = jnp.broadcast_to(
        (jnp.arange(NS * NPASS, dtype=jnp.int32) * NPC)[:, None],
        (NS * NPASS, 16)).astype(jnp.int32)
    osum, osq, ocnt, omin, omax = _sc_aggregate(x, idx, eids, bounds)
    return _epilogue(osum, osq, ocnt, omin, omax, phm_rule, W,
                     bias.reshape(1, D_OUT))


# speedup vs baseline: 72.3992x; 1.3360x over previous
import functools

import numpy as np
import jax
import jax.numpy as jnp
from jax import lax
from jax.experimental import pallas as pl
from jax.experimental.pallas import tpu as pltpu
from jax.experimental.pallas import tpu_sc as plsc

N_NODES = 10000
N_EDGES = 320000
D_IN = 128
PHM_DIM = 4
D_OUT = 128

NC = 2
NS = 16
EH = N_EDGES // NC
NPC = 80
NCHUNKS = N_NODES // NPC
NPASS = (NCHUNKS + NS - 1) // NS
NJUNK = NPC
CH = 16000
NECH = EH // CH
NV = CH // 16
GB = 32
NQ = D_IN // 16

_DEG = np.array([16, 24, 32, 32, 32, 32, 40, 48], dtype=np.float32)
AVG_LOG = float(np.log(_DEG + 1.0).mean())
FBIG = float(3.0e38)


def _sc_aggregate(x, idx, bounds):
    mesh = plsc.VectorSubcoreMesh(core_axis_name="c", subcore_axis_name="s")

    @functools.partial(
        pl.kernel,
        mesh=mesh,
        compiler_params=pltpu.CompilerParams(needs_layout_passes=False),
        out_type=[
            jax.ShapeDtypeStruct((NC, N_NODES, D_IN), jnp.float32),
            jax.ShapeDtypeStruct((NC, N_NODES, D_IN), jnp.float32),
            jax.ShapeDtypeStruct((NC, N_NODES, 16), jnp.float32),
            jax.ShapeDtypeStruct((NC, N_NODES, D_IN), jnp.float32),
            jax.ShapeDtypeStruct((NC, N_NODES, D_IN), jnp.float32),
        ],
        scratch_types=[
            pltpu.VMEM((CH,), jnp.int32),
            pltpu.VMEM((16,), jnp.int32),
            pltpu.VMEM((16,), jnp.int32),
            pltpu.VMEM((CH + GB,), jnp.int32),
            pltpu.VMEM((CH + GB,), jnp.int32),
            pltpu.VMEM((GB, D_IN), jnp.float32),
            pltpu.VMEM((NPC + 1, D_IN), jnp.float32),
            pltpu.VMEM((NPC + 1, D_IN), jnp.float32),
            pltpu.VMEM((NPC + 1, D_IN), jnp.float32),
            pltpu.VMEM((NPC + 1, D_IN), jnp.float32),
            pltpu.VMEM((NPC + 1, 16), jnp.float32),
            pltpu.SemaphoreType.DMA,
        ],
    )
    def k(x_hbm, idx_hbm, bounds_hbm, osum, osq, ocnt, omin, omax,
          ibuf, bbuf, lbuf, listg, listn, xg, accsum, accsq, accmin,
          accmax, acccnt, sem):
        cid = lax.axis_index("c")
        sid = lax.axis_index("s")
        ebase = cid * EH

        zeros16 = jnp.zeros((16,), jnp.float32)
        big16 = jnp.full((16,), FBIG, jnp.float32)
        negbig16 = jnp.full((16,), -FBIG, jnp.float32)
        onef16 = jnp.full((16,), 1.0, jnp.float32)
        one16 = jnp.full((16,), 1, jnp.int32)
        lane = lax.iota(jnp.int32, 16)
        zero16i = jnp.zeros((16,), jnp.int32)
        npc16 = jnp.full((16,), NPC, jnp.int32)
        junk16 = jnp.full((16,), NJUNK, jnp.int32)

        def pass_body(r, _):
            kk = r * NS + sid
            nbase = kk * NPC
            pltpu.sync_copy(bounds_hbm.at[kk], bbuf)
            nb16 = bbuf[pl.ds(0, 16)]
            lim16 = nb16 + npc16

            def arow(q, _):
                for j in range(NQ):
                    sl = pl.ds(j * 16, 16)
                    accsum[q, sl] = zeros16
                    accsq[q, sl] = zeros16
                    accmin[q, sl] = big16
                    accmax[q, sl] = negbig16
                acccnt[q, pl.ds(0, 16)] = zeros16
                return 0
            lax.fori_loop(0, NPC + 1, arow, 0)

            def chunk_body(cc, _):
                e0 = ebase + cc * CH
                pltpu.sync_copy(idx_hbm.at[pl.ds(e0, CH)], ibuf)

                lbuf[pl.ds(0, 16)] = zero16i

                def scan_body(v, _):
                    loff16 = lbuf[pl.ds(0, 16)]
                    iv = ibuf[pl.ds(v * 16, 16)]
                    m = (iv >= nb16) & (iv < lim16)
                    mi = jnp.where(m, one16, zero16i)
                    cs = jnp.cumsum(mi)
                    pos = (loff16 + cs) - one16
                    eg = jnp.full((16,), e0 + v * 16, jnp.int32) + lane
                    plsc.store_scatter(listg, [pos], eg, mask=m)
                    plsc.store_scatter(listn, [pos], iv - nb16, mask=m)
                    lbuf[pl.ds(0, 16)] = (
                        loff16 + plsc.all_reduce_population_count(m))
                    return 0
                lax.fori_loop(0, NV, scan_body, 0)
                nmatch = lbuf[pl.ds(0, 16)][0]

                for t in range(GB // 16):
                    padpos = jnp.full((16,), nmatch + t * 16, jnp.int32) + lane
                    plsc.store_scatter(listg, [padpos], zero16i)
                    plsc.store_scatter(listn, [padpos], junk16)

                def batch_body(b, _):
                    pltpu.async_copy(
                        x_hbm.at[listg.at[pl.ds(b * GB, GB)]], xg, sem,
                    ).wait()

                    def group_body(g, _):
                        nvec = listn[pl.ds(b * GB + g * 16, 16)]
                        for l in range(16):
                            mrow = g * 16 + l
                            n = nvec[l]
                            cv = acccnt[n, pl.ds(0, 16)]
                            acccnt[n, pl.ds(0, 16)] = cv + onef16
                            for j in range(NQ):
                                sl = pl.ds(j * 16, 16)
                                xv = xg[mrow, sl]
                                accsum[n, sl] = accsum[n, sl] + xv
                                accsq[n, sl] = accsq[n, sl] + xv * xv
                                accmin[n, sl] = jnp.minimum(accmin[n, sl], xv)
                                accmax[n, sl] = jnp.maximum(accmax[n, sl], xv)
                        return 0
                    lax.fori_loop(0, GB // 16, group_body, 0)
                    return 0
                lax.fori_loop(0, (nmatch + GB - 1) // GB, batch_body, 0)
                return 0
            lax.fori_loop(0, NECH, chunk_body, 0)

            @pl.when(kk < NCHUNKS)
            def _():
                rows = pl.ds(nbase, NPC)
                src = pl.ds(0, NPC)
                pltpu.sync_copy(accsum.at[src], osum.at[cid, rows])
                pltpu.sync_copy(accsq.at[src], osq.at[cid, rows])
                pltpu.sync_copy(acccnt.at[src], ocnt.at[cid, rows])
                pltpu.sync_copy(accmin.at[src], omin.at[cid, rows])
                pltpu.sync_copy(accmax.at[src], omax.at[cid, rows])
            return 0
        lax.fori_loop(0, NPASS, pass_body, 0)

    return k(x, idx, bounds)


TN = 400


def _epilogue_body(s0_ref, s1_ref, q0_ref, q1_ref, mn0_ref, mn1_ref,
                   mx0_ref, mx1_ref, c0_ref, c1_ref, pr_ref, w_ref,
                   b_ref, o_ref):
    s = s0_ref[...] + s1_ref[...]
    sq = q0_ref[...] + q1_ref[...]
    cnt = (c0_ref[...] + c1_ref[...])[:, 0:1]
    safe = jnp.maximum(cnt, 1.0)
    mean = s / safe
    var = sq / safe - mean * mean
    std = jnp.sqrt(jax.nn.relu(var) + 1e-05)
    has = cnt > 0.0
    mn = jnp.where(has, jnp.minimum(mn0_ref[...], mn1_ref[...]), 0.0)
    mx = jnp.where(has, jnp.maximum(mx0_ref[...], mx1_ref[...]), 0.0)

    parts1 = []
    for p in range(PHM_DIM):
        sl = slice(32 * p, 32 * p + 32)
        parts1 += [mean[:, sl], mn[:, sl], mx[:, sl], std[:, sl]]
    out = jnp.concatenate(parts1, axis=1)

    logdeg = jnp.log(cnt + 1.0)
    amp_s = logdeg / AVG_LOG
    att_s = jnp.where(cnt == 0.0, 1.0,
                      AVG_LOG / jnp.where(cnt == 0.0, 1.0, logdeg))

    parts2 = []
    for p in range(PHM_DIM):
        sl = slice(128 * p, 128 * p + 128)
        op = out[:, sl]
        parts2 += [op, op * amp_s, op * att_s]
    big = jnp.concatenate(parts2, axis=1)

    pr = pr_ref[...]
    w = w_ref[...]
    row_blocks = []
    for p in range(PHM_DIM):
        col_blocks = []
        for j in range(PHM_DIM):
            blk = pr[0, p, j] * w[0]
            for i in range(1, PHM_DIM):
                blk = blk + pr[i, p, j] * w[i]
            col_blocks.append(blk)
        row_blocks.append(jnp.concatenate(col_blocks, axis=1))
    H = jnp.concatenate(row_blocks, axis=0)

    o_ref[...] = (jnp.dot(big, H, preferred_element_type=jnp.float32)
                  + b_ref[...])


def _epilogue(osum, osq, ocnt, omin, omax, phm_rule, W, bias):
    grid = (N_NODES // TN,)
    node_spec = pl.BlockSpec((TN, D_IN), lambda i: (i, 0))
    cnt_spec = pl.BlockSpec((TN, 16), lambda i: (i, 0))
    return pl.pallas_call(
        _epilogue_body,
        grid=grid,
        in_specs=[
            node_spec, node_spec, node_spec, node_spec,
            node_spec, node_spec, node_spec, node_spec,
            cnt_spec, cnt_spec,
            pl.BlockSpec((PHM_DIM, PHM_DIM, PHM_DIM), lambda i: (0, 0, 0)),
            pl.BlockSpec((PHM_DIM, 384, 32), lambda i: (0, 0, 0)),
            pl.BlockSpec((1, D_OUT), lambda i: (0, 0)),
        ],
        out_specs=pl.BlockSpec((TN, D_OUT), lambda i: (i, 0)),
        out_shape=jax.ShapeDtypeStruct((N_NODES, D_OUT), jnp.float32),
    )(osum[0], osum[1], osq[0], osq[1], omin[0], omin[1], omax[0], omax[1],
      ocnt[0], ocnt[1], phm_rule, W, bias)


def kernel(x, idx, phm_rule, W, bias, dim_size):
    bounds = jnp.broadcast_to(
        (jnp.arange(NS * NPASS, dtype=jnp.int32) * NPC)[:, None],
        (NS * NPASS, 16)).astype(jnp.int32)
    osum, osq, ocnt, omin, omax = _sc_aggregate(x, idx, bounds)
    return _epilogue(osum, osq, ocnt, omin, omax, phm_rule, W,
                     bias.reshape(1, D_OUT))
